# baseline (speedup 1.0000x reference)
import functools

import jax
import jax.numpy as jnp
from jax import lax
from jax.experimental import pallas as pl
from jax.experimental.pallas import tpu as pltpu
from jax.experimental.pallas import tpu_sc as plsc

N_USERS = 1000000
EMB_DIM = 64
FEAT_DIM = 64
HID = 256
BATCH = 16384


def _make_sc_gather(V, D, B):
    info = plsc.get_sparse_core_info()
    NC, NS = info.num_cores, info.num_subcores
    NW = NC * NS
    assert B % (8 * NW) == 0
    b_per_w = B // NW
    mesh = plsc.VectorSubcoreMesh(core_axis_name="c", subcore_axis_name="s")

    @functools.partial(
        pl.kernel, mesh=mesh,
        compiler_params=pltpu.CompilerParams(use_tc_tiling_on_sc=False),
        out_type=jax.ShapeDtypeStruct((B, D), jnp.float32),
        scratch_types=[
            pltpu.VMEM((b_per_w,), jnp.int32),
            pltpu.VMEM((b_per_w, D), jnp.float32),
            pltpu.SemaphoreType.DMA,
        ],
    )
    def gather(table_hbm, idx_hbm, out_hbm, idx_v, rows_v, sem):
        wid = lax.axis_index("s") * NC + lax.axis_index("c")
        base = wid * b_per_w
        pltpu.sync_copy(idx_hbm.at[pl.ds(base, b_per_w)], idx_v)
        pltpu.async_copy(table_hbm.at[idx_v], rows_v, sem).wait()
        pltpu.sync_copy(rows_v, out_hbm.at[pl.ds(base, b_per_w)])

    return gather


def _mlp_body(emb, feat, w1a, w1b, b1, w2, b2, w3, b3, out):
    h = jnp.dot(emb[...], w1a[...], preferred_element_type=jnp.float32)
    h += jnp.dot(feat[...], w1b[...], preferred_element_type=jnp.float32)
    h = jnp.maximum(h + b1[...], 0.0)
    h = jnp.maximum(
        jnp.dot(h, w2[...], preferred_element_type=jnp.float32) + b2[...], 0.0)
    out[...] = jnp.dot(h, w3[...], preferred_element_type=jnp.float32) + b3[...]


def _mlp(emb, feat, W1a, W1b, b1, W2, b2, W3, b3, tile):
    B = emb.shape[0]
    grid = (B // tile,)
    full = lambda shape: pl.BlockSpec(shape, lambda i: (0, 0))
    return pl.pallas_call(
        _mlp_body,
        grid=grid,
        in_specs=[
            pl.BlockSpec((tile, EMB_DIM), lambda i: (i, 0)),
            pl.BlockSpec((tile, FEAT_DIM), lambda i: (i, 0)),
            full((EMB_DIM, HID)),
            full((FEAT_DIM, HID)),
            full((1, HID)),
            full((HID, HID)),
            full((1, HID)),
            full((HID, EMB_DIM)),
            full((1, EMB_DIM)),
        ],
        out_specs=pl.BlockSpec((tile, EMB_DIM), lambda i: (i, 0)),
        out_shape=jax.ShapeDtypeStruct((B, EMB_DIM), jnp.float32),
    )(emb, feat, W1a, W1b, b1, W2, b2, W3, b3)


def kernel(user_ids, user_features, table, W1, b1, W2, b2, W3, b3):
    emb = _make_sc_gather(N_USERS, EMB_DIM, BATCH)(table, user_ids.astype(jnp.int32))
    return _mlp(
        emb, user_features,
        W1[:EMB_DIM], W1[EMB_DIM:], b1.reshape(1, HID),
        W2, b2.reshape(1, HID), W3, b3.reshape(1, EMB_DIM),
        tile=2048,
    )


# speedup vs baseline: 2.2907x; 2.2907x over previous
import functools

import jax
import jax.numpy as jnp
from jax import lax
from jax.experimental import pallas as pl
from jax.experimental.pallas import tpu as pltpu
from jax.experimental.pallas import tpu_sc as plsc

N_USERS = 1000000
EMB_DIM = 64
FEAT_DIM = 64
HID = 256
BATCH = 16384
ROWS_PER_TILE = 8


def _make_sc_gather(n_tiles, D, B):
    info = plsc.get_sparse_core_info()
    NC, NS = info.num_cores, info.num_subcores
    NW = NC * NS
    assert B % (8 * NW) == 0
    b_per_w = B // NW
    K = 16
    n_ch = b_per_w // K
    mesh = plsc.VectorSubcoreMesh(core_axis_name="c", subcore_axis_name="s")

    @functools.partial(
        pl.kernel, mesh=mesh,
        out_type=jax.ShapeDtypeStruct((B, D), jnp.float32),
        scratch_types=[
            pltpu.VMEM((b_per_w,), jnp.int32),
            pltpu.VMEM((b_per_w, D), jnp.float32),
            pltpu.SemaphoreType.DMA,
        ],
    )
    def gather(table_hbm, idx_hbm, out_hbm, ids_v, rows_v, sem):
        wid = lax.axis_index("s") * NC + lax.axis_index("c")
        base = wid * b_per_w
        pltpu.sync_copy(idx_hbm.at[pl.ds(base, b_per_w)], ids_v)

        def chunk_body(c, carry):
            idv = ids_v[pl.ds(c * K, 16)]
            copies = []
            for jj in range(K):
                t = lax.shift_right_logical(idv[jj], 3)
                r = lax.rem(idv[jj], ROWS_PER_TILE)
                copies.append(pltpu.async_copy(
                    table_hbm.at[t, r], rows_v.at[c * K + jj], sem))
            for cp in copies:
                cp.wait()
            return carry

        lax.fori_loop(0, n_ch, chunk_body, 0)
        pltpu.sync_copy(rows_v, out_hbm.at[pl.ds(base, b_per_w)])

    return gather


def _mlp_body(emb, feat, w1a, w1b, b1, w2, b2, w3, b3, out):
    h = jnp.dot(emb[...], w1a[...], preferred_element_type=jnp.float32)
    h += jnp.dot(feat[...], w1b[...], preferred_element_type=jnp.float32)
    h = jnp.maximum(h + b1[...], 0.0)
    h = jnp.maximum(
        jnp.dot(h, w2[...], preferred_element_type=jnp.float32) + b2[...], 0.0)
    out[...] = jnp.dot(h, w3[...], preferred_element_type=jnp.float32) + b3[...]


def _mlp(emb, feat, W1a, W1b, b1, W2, b2, W3, b3, tile):
    B = emb.shape[0]
    grid = (B // tile,)
    full = lambda shape: pl.BlockSpec(shape, lambda i: (0, 0))
    return pl.pallas_call(
        _mlp_body,
        grid=grid,
        in_specs=[
            pl.BlockSpec((tile, EMB_DIM), lambda i: (i, 0)),
            pl.BlockSpec((tile, FEAT_DIM), lambda i: (i, 0)),
            full((EMB_DIM, HID)),
            full((FEAT_DIM, HID)),
            full((1, HID)),
            full((HID, HID)),
            full((1, HID)),
            full((HID, EMB_DIM)),
            full((1, EMB_DIM)),
        ],
        out_specs=pl.BlockSpec((tile, EMB_DIM), lambda i: (i, 0)),
        out_shape=jax.ShapeDtypeStruct((B, EMB_DIM), jnp.float32),
    )(emb, feat, W1a, W1b, b1, W2, b2, W3, b3)


def kernel(user_ids, user_features, table, W1, b1, W2, b2, W3, b3):
    table3 = table.reshape(N_USERS // ROWS_PER_TILE, ROWS_PER_TILE, EMB_DIM)
    emb = _make_sc_gather(N_USERS // ROWS_PER_TILE, EMB_DIM, BATCH)(
        table3, user_ids.astype(jnp.int32))
    return _mlp(
        emb, user_features,
        W1[:EMB_DIM], W1[EMB_DIM:], b1.reshape(1, HID),
        W2, b2.reshape(1, HID), W3, b3.reshape(1, EMB_DIM),
        tile=2048,
    )
